# baseline (speedup 1.0000x reference)
import functools

import jax
import jax.numpy as jnp
from jax import lax
from jax.experimental import pallas as pl
from jax.experimental.pallas import tpu as pltpu
from jax.experimental.pallas import tpu_sc as plsc


def _build_lookup(B1, B2, V, D):
    info = plsc.get_sparse_core_info()
    NC, NS = info.num_cores, info.num_subcores
    NW = NC * NS
    DT = D // 8
    CT = B1 // 128
    n_pairs = B2 * CT
    p_per_w = n_pairs // NW
    idx_per_w = p_per_w * 128
    out_words = B2 * DT * CT * 1024
    mesh = plsc.VectorSubcoreMesh(core_axis_name="c", subcore_axis_name="s")

    @functools.partial(
        pl.kernel,
        mesh=mesh,
        out_type=jax.ShapeDtypeStruct((out_words,), jnp.float32),
        compiler_params=pltpu.CompilerParams(use_tc_tiling_on_sc=False,
                                             needs_layout_passes=False),
        scratch_types=[
            pltpu.VMEM((idx_per_w,), jnp.int32),
            pltpu.VMEM((128, D), jnp.float32),
            pltpu.VMEM((128, D), jnp.float32),
            pltpu.VMEM((D * 128,), jnp.float32),
            pltpu.VMEM((D * 128,), jnp.float32),
            pltpu.SemaphoreType.DMA,
            pltpu.SemaphoreType.DMA,
            pltpu.SemaphoreType.DMA,
            pltpu.SemaphoreType.DMA,
        ],
    )
    def lookup(idx_hbm, table_hbm, out_hbm, idx_v, rows0, rows1, t0, t1,
               sem_g0, sem_g1, sem_s0, sem_s1):
        wid = lax.axis_index("s") * NC + lax.axis_index("c")
        p_base = wid * p_per_w
        pltpu.sync_copy(idx_hbm.at[pl.ds(pl.multiple_of(p_base * 128, 8),
                                         idx_per_w)], idx_v)

        rows = (rows0, rows1)
        tbuf = (t0, t1)
        sem_g = (sem_g0, sem_g1)
        sem_s = (sem_s0, sem_s1)

        iota = lax.iota(jnp.int32, 16)
        flat_base = [(16 * j + iota) * 128 for j in range(D // 16)]

        def fire_gather(q, b):
            idx_slice = idx_v.at[pl.ds(pl.multiple_of(q * 128, 8), 128)]
            pltpu.async_copy(table_hbm.at[idx_slice], rows[b], sem_g[b])

        def wait_gather(q, b):
            idx_slice = idx_v.at[pl.ds(pl.multiple_of(q * 128, 8), 128)]
            pltpu.make_async_copy(table_hbm.at[idx_slice], rows[b],
                                  sem_g[b]).wait()

        def fire_store(q, b):
            p = p_base + q
            b2 = p // CT
            ct = p % CT
            for dt in range(DT):
                addr = ((b2 * DT + dt) * CT + ct) * 1024
                pltpu.async_copy(tbuf[b].at[pl.ds(dt * 1024, 1024)],
                                 out_hbm.at[pl.ds(addr, 1024)], sem_s[b])

        def wait_store(q, b):
            p = p_base + q
            b2 = p // CT
            ct = p % CT
            addr = (b2 * DT * CT + ct) * 1024
            pltpu.make_async_copy(tbuf[b],
                                  out_hbm.at[pl.ds(addr, D * 128)],
                                  sem_s[b]).wait()

        def transpose(b):
            src, dst = rows[b], tbuf[b]

            def kbody(k, carry):
                for j in range(D // 16):
                    x = src[k, pl.ds(16 * j, 16)]
                    plsc.store_scatter(dst, [flat_base[j] + k], x)
                return carry

            lax.fori_loop(0, 128, kbody, 0, unroll=8)

        fire_gather(0, 0)
        fire_gather(1, 1)
        for q in (0, 1):
            b = q & 1
            wait_gather(q, b)
            transpose(b)
            fire_store(q, b)
            fire_gather(q + 2, b)

        def body(i, carry):
            q = 2 + 2 * i
            for b in (0, 1):
                wait_gather(q + b, b)
                wait_store(q + b - 2, b)
                transpose(b)
                fire_store(q + b, b)
                fire_gather(q + b + 2, b)
            return carry

        lax.fori_loop(0, (p_per_w - 4) // 2, body, 0)

        for q in (p_per_w - 2, p_per_w - 1):
            b = q & 1
            wait_gather(q, b)
            wait_store(q - 2, b)
            transpose(b)
            fire_store(q, b)
        wait_store(p_per_w - 2, 0)
        wait_store(p_per_w - 1, 1)

    return lookup


def kernel(token_ids, weight):
    V, D = weight.shape
    B1, B2 = token_ids.shape
    idx_flat = token_ids.astype(jnp.int32).T.reshape(B1 * B2)
    out1d = _build_lookup(B1, B2, V, D)(idx_flat, weight)
    out5 = out1d.reshape(B2, D // 8, B1 // 128, 8, 128)
    return out5.transpose(2, 4, 0, 1, 3).reshape(B1, B2, D)


# speedup vs baseline: 1.2204x; 1.2204x over previous
import functools

import jax
import jax.numpy as jnp
from jax import lax
from jax.experimental import pallas as pl
from jax.experimental.pallas import tpu as pltpu
from jax.experimental.pallas import tpu_sc as plsc


def _build_lookup(B1, B2, V, D):
    info = plsc.get_sparse_core_info()
    NC, NS = info.num_cores, info.num_subcores
    NW = NC * NS
    DT = D // 8
    CT = B1 // 128
    n_pairs = B2 * CT
    p_per_w = n_pairs // NW
    idx_per_w = p_per_w * 128
    out_words = B2 * DT * CT * 1024
    mesh = plsc.VectorSubcoreMesh(core_axis_name="c", subcore_axis_name="s")

    @functools.partial(
        pl.kernel,
        mesh=mesh,
        out_type=jax.ShapeDtypeStruct((out_words,), jnp.float32),
        compiler_params=pltpu.CompilerParams(use_tc_tiling_on_sc=False,
                                             needs_layout_passes=False,
                                             disable_bounds_checks=True),
        scratch_types=[
            pltpu.VMEM((idx_per_w,), jnp.int32),
            pltpu.VMEM((128, D), jnp.float32),
            pltpu.VMEM((128, D), jnp.float32),
            pltpu.VMEM((D * 128,), jnp.float32),
            pltpu.VMEM((D * 128,), jnp.float32),
            pltpu.SemaphoreType.DMA,
            pltpu.SemaphoreType.DMA,
            pltpu.SemaphoreType.DMA,
            pltpu.SemaphoreType.DMA,
        ],
    )
    def lookup(idx_hbm, table_hbm, out_hbm, idx_v, rows0, rows1, t0, t1,
               sem_g0, sem_g1, sem_s0, sem_s1):
        wid = lax.axis_index("s") * NC + lax.axis_index("c")
        p_base = wid * p_per_w
        pltpu.sync_copy(idx_hbm.at[pl.ds(pl.multiple_of(p_base * 128, 8),
                                         idx_per_w)], idx_v)

        rows = (rows0, rows1)
        tbuf = (t0, t1)
        sem_g = (sem_g0, sem_g1)
        sem_s = (sem_s0, sem_s1)

        iota = lax.iota(jnp.int32, 16)
        flat_base = [(16 * j + iota) * 128 for j in range(D // 16)]

        def fire_gather(q, b):
            idx_slice = idx_v.at[pl.ds(pl.multiple_of(q * 128, 8), 128)]
            pltpu.async_copy(table_hbm.at[idx_slice], rows[b], sem_g[b])

        def wait_gather(q, b):
            idx_slice = idx_v.at[pl.ds(pl.multiple_of(q * 128, 8), 128)]
            pltpu.make_async_copy(table_hbm.at[idx_slice], rows[b],
                                  sem_g[b]).wait()

        def fire_store(q, b):
            p = p_base + q
            b2 = p // CT
            ct = p % CT
            for dt in range(DT):
                addr = ((b2 * DT + dt) * CT + ct) * 1024
                pltpu.async_copy(tbuf[b].at[pl.ds(dt * 1024, 1024)],
                                 out_hbm.at[pl.ds(addr, 1024)], sem_s[b])

        def wait_store(q, b):
            p = p_base + q
            b2 = p // CT
            ct = p % CT
            addr = (b2 * DT * CT + ct) * 1024
            pltpu.make_async_copy(tbuf[b],
                                  out_hbm.at[pl.ds(addr, D * 128)],
                                  sem_s[b]).wait()

        def transpose(b):
            src, dst = rows[b], tbuf[b]

            @plsc.parallel_loop(0, 128, unroll=8)
            def kbody(k):
                for j in range(D // 16):
                    x = src[k, pl.ds(16 * j, 16)]
                    plsc.store_scatter(dst, [flat_base[j] + k], x)

        fire_gather(0, 0)
        fire_gather(1, 1)
        for q in (0, 1):
            b = q & 1
            wait_gather(q, b)
            transpose(b)
            fire_store(q, b)
            fire_gather(q + 2, b)

        def body(i, carry):
            q = 2 + 2 * i
            for b in (0, 1):
                wait_gather(q + b, b)
                wait_store(q + b - 2, b)
                transpose(b)
                fire_store(q + b, b)
                fire_gather(q + b + 2, b)
            return carry

        lax.fori_loop(0, (p_per_w - 4) // 2, body, 0)

        for q in (p_per_w - 2, p_per_w - 1):
            b = q & 1
            wait_gather(q, b)
            wait_store(q - 2, b)
            transpose(b)
            fire_store(q, b)
        wait_store(p_per_w - 2, 0)
        wait_store(p_per_w - 1, 1)

    return lookup


def kernel(token_ids, weight):
    V, D = weight.shape
    B1, B2 = token_ids.shape
    idx_flat = token_ids.astype(jnp.int32).T.reshape(B1 * B2)
    out1d = _build_lookup(B1, B2, V, D)(idx_flat, weight)
    out5 = out1d.reshape(B2, D // 8, B1 // 128, 8, 128)
    return out5.transpose(2, 4, 0, 1, 3).reshape(B1, B2, D)


# speedup vs baseline: 2.0219x; 1.6568x over previous
import functools

import jax
import jax.numpy as jnp
from jax import lax
from jax.experimental import pallas as pl
from jax.experimental.pallas import tpu as pltpu
from jax.experimental.pallas import tpu_sc as plsc


def _build_lookup(B1, B2, V, D):
    info = plsc.get_sparse_core_info()
    NC, NS = info.num_cores, info.num_subcores
    NW = NC * NS
    DT = D // 8
    CT = B1 // 128
    n_pairs = B2 * CT
    p_per_w = n_pairs // NW
    idx_per_w = p_per_w * 128
    out_words = B2 * DT * CT * 1024
    mesh = plsc.VectorSubcoreMesh(core_axis_name="c", subcore_axis_name="s")

    @functools.partial(
        pl.kernel,
        mesh=mesh,
        out_type=jax.ShapeDtypeStruct((B2, DT, CT, 8, 128), jnp.float32),
        compiler_params=pltpu.CompilerParams(use_tc_tiling_on_sc=False,
                                             needs_layout_passes=False,
                                             disable_bounds_checks=True),
        scratch_types=[
            pltpu.VMEM((idx_per_w,), jnp.int32),
            pltpu.VMEM((128, D), jnp.float32),
            pltpu.VMEM((128, D), jnp.float32),
            pltpu.VMEM((DT, 8, 145), jnp.float32),
            pltpu.VMEM((DT, 8, 145), jnp.float32),
            pltpu.SemaphoreType.DMA,
            pltpu.SemaphoreType.DMA,
            pltpu.SemaphoreType.DMA,
            pltpu.SemaphoreType.DMA,
        ],
    )
    def lookup(idx_hbm, table_hbm, out_hbm, idx_v, rows0, rows1, t0, t1,
               sem_g0, sem_g1, sem_s0, sem_s1):
        wid = lax.axis_index("s") * NC + lax.axis_index("c")
        p_base = wid * p_per_w
        pltpu.sync_copy(idx_hbm.at[pl.ds(pl.multiple_of(p_base * 128, 8),
                                         idx_per_w)], idx_v)

        rows = (rows0, rows1)
        tbuf = (t0, t1)
        sem_g = (sem_g0, sem_g1)
        sem_s = (sem_s0, sem_s1)

        iota = lax.iota(jnp.int32, 16)
        dt_vecs = [(16 * j + iota) // 8 for j in range(D // 16)]
        r_vecs = [(16 * j + iota) % 8 for j in range(D // 16)]

        def fire_gather(q, b):
            idx_slice = idx_v.at[pl.ds(pl.multiple_of(q * 128, 8), 128)]
            pltpu.async_copy(table_hbm.at[idx_slice], rows[b], sem_g[b])

        def wait_gather(q, b):
            idx_slice = idx_v.at[pl.ds(pl.multiple_of(q * 128, 8), 128)]
            pltpu.make_async_copy(table_hbm.at[idx_slice], rows[b],
                                  sem_g[b]).wait()

        def fire_store(q, b):
            p = p_base + q
            pltpu.async_copy(tbuf[b].at[:, :, pl.ds(0, 128)],
                             out_hbm.at[p // CT, :, p % CT], sem_s[b])

        def wait_store(q, b):
            p = p_base + q
            pltpu.make_async_copy(tbuf[b].at[:, :, pl.ds(0, 128)],
                                  out_hbm.at[p // CT, :, p % CT],
                                  sem_s[b]).wait()

        def transpose(b):
            src, dst = rows[b], tbuf[b]

            @plsc.parallel_loop(0, 128, unroll=8)
            def kbody(k):
                l_vec = jnp.full((16,), k, jnp.int32)
                for j in range(D // 16):
                    x = src[k, pl.ds(16 * j, 16)]
                    plsc.store_scatter(dst, [dt_vecs[j], r_vecs[j], l_vec], x)

        fire_gather(0, 0)
        fire_gather(1, 1)
        for q in (0, 1):
            b = q & 1
            wait_gather(q, b)
            transpose(b)
            fire_store(q, b)
            fire_gather(q + 2, b)

        def body(i, carry):
            q = 2 + 2 * i
            for b in (0, 1):
                wait_gather(q + b, b)
                wait_store(q + b - 2, b)
                transpose(b)
                fire_store(q + b, b)
                fire_gather(q + b + 2, b)
            return carry

        lax.fori_loop(0, (p_per_w - 4) // 2, body, 0)

        for q in (p_per_w - 2, p_per_w - 1):
            b = q & 1
            wait_gather(q, b)
            wait_store(q - 2, b)
            transpose(b)
            fire_store(q, b)
        wait_store(p_per_w - 2, 0)
        wait_store(p_per_w - 1, 1)

    return lookup


def kernel(token_ids, weight):
    V, D = weight.shape
    B1, B2 = token_ids.shape
    idx_flat = token_ids.astype(jnp.int32).T.reshape(B1 * B2)
    out5 = _build_lookup(B1, B2, V, D)(idx_flat, weight)
    return out5.transpose(2, 4, 0, 1, 3).reshape(B1, B2, D)
